# Initial kernel scaffold; baseline (speedup 1.0000x reference)
import jax
import jax.numpy as jnp
from jax.experimental import pallas as pl


def kernel(metric, x):
    raise NotImplementedError("write your pallas kernel here")


# speedup vs baseline: 1.1579x; 1.1579x over previous
import functools
import math

import jax
import jax.numpy as jnp
from jax.experimental import pallas as pl
from jax.experimental.pallas import tpu as pltpu

_INTERP = False

B, T, C = 4, 4096, 1024
T2 = T // 2
R = 0.95
K = math.floor(T - T * R)


def _scores_body(TJ, a_ref, b_ref, na_ref, nb_ref, max_ref, idx_ref):
    j = pl.program_id(2)
    a = a_ref[0, :, 0:C]
    b = b_ref[0, :, C:2 * C]
    an = a / na_ref[0, :, 0:1]
    bn = b / nb_ref[0, :, 1:2]
    s = jax.lax.dot_general(
        an, bn, (((1,), (1,)), ((), ())),
        preferred_element_type=jnp.float32,
    )
    m = jnp.max(s, axis=-1, keepdims=True)
    col = jax.lax.broadcasted_iota(jnp.int32, s.shape, 1) + j * TJ
    idx = jnp.min(jnp.where(s == m, col, T2), axis=-1, keepdims=True)

    @pl.when(j == 0)
    def _():
        max_ref[...] = m[None]
        idx_ref[...] = idx[None]

    @pl.when(j > 0)
    def _():
        cur_m = max_ref[...]
        cur_i = idx_ref[...]
        upd = m[None] > cur_m
        max_ref[...] = jnp.where(upd, m[None], cur_m)
        idx_ref[...] = jnp.where(upd, idx[None], cur_i)


def _node_max_idx(metric):
    TI = 512
    TJ = 512
    ni = T2 // TI
    nj = T2 // TJ
    m3 = metric.reshape(B, T2, 2 * C)
    n3 = jnp.linalg.norm(metric, axis=-1, keepdims=True).reshape(B, T2, 2)
    grid = (B, ni, nj)
    out = pl.pallas_call(
        functools.partial(_scores_body, TJ),
        grid=grid,
        in_specs=[
            pl.BlockSpec((1, TI, 2 * C), lambda b, i, j: (b, i, 0)),
            pl.BlockSpec((1, TJ, 2 * C), lambda b, i, j: (b, j, 0)),
            pl.BlockSpec((1, TI, 2), lambda b, i, j: (b, i, 0)),
            pl.BlockSpec((1, TJ, 2), lambda b, i, j: (b, j, 0)),
        ],
        out_specs=[
            pl.BlockSpec((1, TI, 1), lambda b, i, j: (b * ni + i, 0, 0)),
            pl.BlockSpec((1, TI, 1), lambda b, i, j: (b * ni + i, 0, 0)),
        ],
        out_shape=[
            jax.ShapeDtypeStruct((B * ni, TI, 1), jnp.float32),
            jax.ShapeDtypeStruct((B * ni, TI, 1), jnp.int32),
        ],
        interpret=_INTERP,
    )(m3, m3, n3, n3)
    node_max = out[0].reshape(B, T2)
    node_idx = out[1].reshape(B, T2)
    return node_max, node_idx


def kernel(metric, x):
    node_max, node_idx = _node_max_idx(metric)

    edge_idx = jnp.argsort(-node_max, axis=-1)
    unm_idx = edge_idx[:, K:]
    src_idx = edge_idx[:, :K]
    dst_idx = jnp.take_along_axis(node_idx, src_idx, axis=-1)
    bidx = jnp.arange(B)[:, None]

    def merge_sum(v):
        src = v[:, ::2, :]
        dst = v[:, 1::2, :]
        unm = jnp.take_along_axis(src, unm_idx[:, :, None], axis=1)
        s = jnp.take_along_axis(src, src_idx[:, :, None], axis=1)
        dst = dst.at[bidx, dst_idx].add(s)
        return jnp.concatenate([unm, dst], axis=1)

    size = jnp.ones((B, T, 1), dtype=x.dtype)
    xm = merge_sum(x * size)
    sm = merge_sum(size)
    return xm / sm


# speedup vs baseline: 1.5664x; 1.3527x over previous
import functools
import math

import jax
import jax.numpy as jnp
from jax import lax
from jax.experimental import pallas as pl
from jax.experimental.pallas import tpu as pltpu
from jax.experimental.pallas import tpu_sc as plsc

_INTERP = False

B, T, C = 4, 4096, 1024
T2 = T // 2
R = 0.95
K = math.floor(T - T * R)
UNM = T2 - K
TOUT = T2 + UNM


def _scores_body(TJ, a_ref, b_ref, na_ref, nb_ref, max_ref, idx_ref):
    j = pl.program_id(2)
    a = a_ref[0, :, 0:C]
    b = b_ref[0, :, C:2 * C]
    an = a / na_ref[0, :, 0:1]
    bn = b / nb_ref[0, :, 1:2]
    s = jax.lax.dot_general(
        an, bn, (((1,), (1,)), ((), ())),
        preferred_element_type=jnp.float32,
    )
    m = jnp.max(s, axis=-1, keepdims=True)
    col = jax.lax.broadcasted_iota(jnp.int32, s.shape, 1) + j * TJ
    idx = jnp.min(jnp.where(s == m, col, T2), axis=-1, keepdims=True)

    @pl.when(j == 0)
    def _():
        max_ref[...] = m[None]
        idx_ref[...] = idx[None]

    @pl.when(j > 0)
    def _():
        cur_m = max_ref[...]
        cur_i = idx_ref[...]
        upd = m[None] > cur_m
        max_ref[...] = jnp.where(upd, m[None], cur_m)
        idx_ref[...] = jnp.where(upd, idx[None], cur_i)


def _node_max_idx(metric):
    TI = 512
    TJ = 512
    ni = T2 // TI
    nj = T2 // TJ
    m3 = metric.reshape(B, T2, 2 * C)
    n3 = jnp.linalg.norm(metric, axis=-1, keepdims=True).reshape(B, T2, 2)
    grid = (B, ni, nj)
    out = pl.pallas_call(
        functools.partial(_scores_body, TJ),
        grid=grid,
        in_specs=[
            pl.BlockSpec((1, TI, 2 * C), lambda b, i, j: (b, i, 0)),
            pl.BlockSpec((1, TJ, 2 * C), lambda b, i, j: (b, j, 0)),
            pl.BlockSpec((1, TI, 2), lambda b, i, j: (b, i, 0)),
            pl.BlockSpec((1, TJ, 2), lambda b, i, j: (b, j, 0)),
        ],
        out_specs=[
            pl.BlockSpec((1, TI, 1), lambda b, i, j: (b * ni + i, 0, 0)),
            pl.BlockSpec((1, TI, 1), lambda b, i, j: (b * ni + i, 0, 0)),
        ],
        out_shape=[
            jax.ShapeDtypeStruct((B * ni, TI, 1), jnp.float32),
            jax.ShapeDtypeStruct((B * ni, TI, 1), jnp.int32),
        ],
        interpret=_INTERP,
    )(m3, m3, n3, n3)
    node_max = out[0].reshape(B, T2)
    node_idx = out[1].reshape(B, T2)
    return node_max, node_idx


def _rank_body(ni2, TI2, c_ref, r_ref, o_ref):
    i = pl.program_id(1)
    vi = c_ref[0]
    vj = r_ref[0]
    gt = (vj > vi)
    jj = jax.lax.broadcasted_iota(jnp.int32, (TI2, T2), 1)
    ii = jax.lax.broadcasted_iota(jnp.int32, (TI2, T2), 0) + i * TI2
    tie = (vj == vi) & (jj < ii)
    rank = jnp.sum(gt.astype(jnp.int32) + tie.astype(jnp.int32),
                   axis=-1, keepdims=True)
    o_ref[...] = rank[None]


def _rank_tc(node_max):
    TI2 = 128
    ni2 = T2 // TI2
    nm_c = node_max.reshape(B * ni2, TI2, 1)
    nm_r = node_max.reshape(B, 1, T2)
    rank = pl.pallas_call(
        functools.partial(_rank_body, ni2, TI2),
        grid=(B, ni2),
        in_specs=[
            pl.BlockSpec((1, TI2, 1), lambda b, i: (b * ni2 + i, 0, 0)),
            pl.BlockSpec((1, 1, T2), lambda b, i: (b, 0, 0)),
        ],
        out_specs=pl.BlockSpec((1, TI2, 1), lambda b, i: (b * ni2 + i, 0, 0)),
        out_shape=jax.ShapeDtypeStruct((B * ni2, TI2, 1), jnp.int32),
        interpret=_INTERP,
    )(nm_c, nm_r)
    return rank.reshape(B, T2)


def _merge_body(rank_hbm, nidx_hbm, xq_hbm, out_hbm,
                rank_v, nidx_v, edge_v, edgeu_v, cnt_v,
                dstall_v, srcall_v, mpos_v, msrc_v, midx_v,
                uidx_v, widx_v, didx_v, urows_v, dbuf_v, mrows_v, sem):
    c = lax.axis_index("c")
    s = lax.axis_index("s")
    lanes = lax.iota(jnp.int32, 16)
    ones16 = jnp.ones((16,), jnp.float32)
    r0 = s * 128

    for bi in range(2):
        b = c * 2 + bi
        pltpu.sync_copy(rank_hbm.at[b], rank_v)
        pltpu.sync_copy(nidx_hbm.at[b], nidx_v)

        def inv_body(p, carry):
            rk = rank_v[pl.ds(p * 16, 16)]
            iv = p * 16 + lanes
            plsc.store_scatter(edge_v, [rk], iv, mask=rk < 208)
            plsc.store_scatter(edgeu_v, [rk - K], iv, mask=rk >= K)
            cnt_v[pl.ds(p * 16, 16)] = jnp.zeros((16,), jnp.float32)
            return carry

        lax.fori_loop(0, T2 // 16, inv_body, 0)

        for q in range(13):
            e = edge_v[pl.ds(q * 16, 16)]
            dpos = plsc.load_gather(nidx_v, [e])
            msk = (q * 16 + lanes) < K
            plsc.addupdate_scatter(cnt_v, [dpos], ones16, mask=msk)
            dstall_v[pl.ds(q * 16, 16)] = jnp.where(msk, dpos, T2)
            srcall_v[pl.ds(q * 16, 16)] = (b * T + 2 * e) * 4

        def scan_body(m, nm):
            dsp = plsc.load_gather(dstall_v, [jnp.full((16,), m, jnp.int32)])
            ssp = plsc.load_gather(srcall_v, [jnp.full((16,), m, jnp.int32)])
            dsc = jnp.sum(jnp.where(lanes == 0, dsp, 0))
            inwin = (dsc >= r0) & (dsc < r0 + 128)
            mk = (lanes == 0) & inwin
            plsc.store_scatter(mpos_v, [jnp.full((16,), nm, jnp.int32)], dsp,
                               mask=mk)
            plsc.store_scatter(msrc_v, [jnp.full((16,), nm, jnp.int32)], ssp,
                               mask=mk)
            return nm + inwin.astype(jnp.int32)

        nmatch = lax.fori_loop(0, K, scan_body, jnp.int32(0))
        plsc.store_scatter(msrc_v, [nmatch + lanes],
                           jnp.zeros((16,), jnp.int32))
        plsc.store_scatter(mpos_v, [nmatch + lanes],
                           jnp.full((16,), r0, jnp.int32))

        for it in range(8):
            t = it * 16 + s

            @pl.when(t < 116)
            def _():
                for q in range(4):
                    p = t * 16 + (q * 16 + lanes) // 4
                    pc = jnp.minimum(p, UNM - 1)
                    e = plsc.load_gather(edgeu_v, [pc])
                    quarter = lanes % 4
                    uidx_v[pl.ds(q * 16, 16)] = (b * T + 2 * e) * 4 + quarter
                    widx_v[pl.ds(q * 16, 16)] = (b * TOUT + pc) * 4 + quarter
                pltpu.async_copy(xq_hbm.at[uidx_v], urows_v, sem).wait()
                pltpu.async_copy(urows_v, out_hbm.at[widx_v], sem).wait()

        for h in range(4):
            for q in range(8):
                j = r0 + q * 16 + lanes
                didx_v[pl.ds(q * 16, 16)] = (b * T + 2 * j + 1) * 4 + h
            pltpu.async_copy(xq_hbm.at[didx_v], dbuf_v, sem).wait()

            def shift_body(cq, carry):
                midx_v[pl.ds(cq * 16, 16)] = (
                    msrc_v[pl.ds(cq * 16, 16)] + h)
                return carry

            lax.fori_loop(0, 16, shift_body, 0)

            def grp_body(g, carry):
                pltpu.async_copy(
                    xq_hbm.at[midx_v.at[pl.ds(g * 16, 16)]], mrows_v,
                    sem).wait()

                def lane_body(l, carry2):
                    kk = g * 16 + l

                    @pl.when(kk < nmatch)
                    def _():
                        psp = plsc.load_gather(
                            mpos_v, [jnp.full((16,), kk, jnp.int32)])
                        row = jnp.sum(jnp.where(lanes == 0, psp, 0)) - r0
                        for cc in range(16):
                            dbuf_v[row, pl.ds(cc * 16, 16)] = (
                                dbuf_v[row, pl.ds(cc * 16, 16)]
                                + mrows_v[l, pl.ds(cc * 16, 16)])
                    return carry2

                lax.fori_loop(0, 16, lane_body, 0)
                return carry

            ngrp = (nmatch + 15) // 16
            lax.fori_loop(0, ngrp, grp_body, 0)

            def div_body(j, carry):
                csp = plsc.load_gather(
                    cnt_v, [jnp.full((16,), r0 + j, jnp.int32)])
                d = csp + 1.0
                for cc in range(16):
                    dbuf_v[j, pl.ds(cc * 16, 16)] = (
                        dbuf_v[j, pl.ds(cc * 16, 16)] / d)
                return carry

            lax.fori_loop(0, 128, div_body, 0)
            for q in range(8):
                j = r0 + q * 16 + lanes
                widx8 = (b * TOUT + UNM + j) * 4 + h
                didx_v[pl.ds(q * 16, 16)] = widx8
            pltpu.async_copy(dbuf_v, out_hbm.at[didx_v], sem).wait()


def _merge_sc(rank, node_idx, x):
    xq = x.reshape(B * T * 4, C // 4)
    mesh = plsc.VectorSubcoreMesh(core_axis_name="c", subcore_axis_name="s")
    out = pl.kernel(
        _merge_body,
        out_type=jax.ShapeDtypeStruct((B * TOUT * 4, C // 4), jnp.float32),
        mesh=mesh,
        scratch_types=[
            pltpu.VMEM((T2,), jnp.int32),
            pltpu.VMEM((T2,), jnp.int32),
            pltpu.VMEM((256,), jnp.int32),
            pltpu.VMEM((1920,), jnp.int32),
            pltpu.VMEM((T2,), jnp.float32),
            pltpu.VMEM((208,), jnp.int32),
            pltpu.VMEM((208,), jnp.int32),
            pltpu.VMEM((256,), jnp.int32),
            pltpu.VMEM((256,), jnp.int32),
            pltpu.VMEM((256,), jnp.int32),
            pltpu.VMEM((64,), jnp.int32),
            pltpu.VMEM((64,), jnp.int32),
            pltpu.VMEM((128,), jnp.int32),
            pltpu.VMEM((64, 256), jnp.float32),
            pltpu.VMEM((128, 256), jnp.float32),
            pltpu.VMEM((16, 256), jnp.float32),
            pltpu.SemaphoreType.DMA,
        ],
        compiler_params=pltpu.CompilerParams(needs_layout_passes=False),
    )(rank, node_idx, xq)
    return out.reshape(B, TOUT, C)


def kernel(metric, x):
    node_max, node_idx = _node_max_idx(metric)
    rank = _rank_tc(node_max)
    return _merge_sc(rank, node_idx, x)


# speedup vs baseline: 1.6551x; 1.0566x over previous
import functools
import math

import jax
import jax.numpy as jnp
from jax import lax
from jax.experimental import pallas as pl
from jax.experimental.pallas import tpu as pltpu
from jax.experimental.pallas import tpu_sc as plsc

_INTERP = False

B, T, C = 4, 4096, 1024
T2 = T // 2
R = 0.95
K = math.floor(T - T * R)
UNM = T2 - K
TOUT = T2 + UNM


def _scores_body(TJ, a_ref, b_ref, na_ref, nb_ref, max_ref, idx_ref):
    j = pl.program_id(2)
    a = a_ref[0, :, 0:C]
    b = b_ref[0, :, C:2 * C]
    an = a / na_ref[0, :, 0:1]
    bn = b / nb_ref[0, :, 1:2]
    s = jax.lax.dot_general(
        an, bn, (((1,), (1,)), ((), ())),
        preferred_element_type=jnp.float32,
    )
    m = jnp.max(s, axis=-1, keepdims=True)
    col = jax.lax.broadcasted_iota(jnp.int32, s.shape, 1) + j * TJ
    idx = jnp.min(jnp.where(s == m, col, T2), axis=-1, keepdims=True)

    @pl.when(j == 0)
    def _():
        max_ref[...] = m[None]
        idx_ref[...] = idx[None]

    @pl.when(j > 0)
    def _():
        cur_m = max_ref[...]
        cur_i = idx_ref[...]
        upd = m[None] > cur_m
        max_ref[...] = jnp.where(upd, m[None], cur_m)
        idx_ref[...] = jnp.where(upd, idx[None], cur_i)


def _node_max_idx(metric):
    TI = 512
    TJ = 512
    ni = T2 // TI
    nj = T2 // TJ
    m3 = metric.reshape(B, T2, 2 * C)
    n3 = jnp.linalg.norm(metric, axis=-1, keepdims=True).reshape(B, T2, 2)
    grid = (B, ni, nj)
    out = pl.pallas_call(
        functools.partial(_scores_body, TJ),
        grid=grid,
        in_specs=[
            pl.BlockSpec((1, TI, 2 * C), lambda b, i, j: (b, i, 0)),
            pl.BlockSpec((1, TJ, 2 * C), lambda b, i, j: (b, j, 0)),
            pl.BlockSpec((1, TI, 2), lambda b, i, j: (b, i, 0)),
            pl.BlockSpec((1, TJ, 2), lambda b, i, j: (b, j, 0)),
        ],
        out_specs=[
            pl.BlockSpec((1, TI, 1), lambda b, i, j: (b * ni + i, 0, 0)),
            pl.BlockSpec((1, TI, 1), lambda b, i, j: (b * ni + i, 0, 0)),
        ],
        out_shape=[
            jax.ShapeDtypeStruct((B * ni, TI, 1), jnp.float32),
            jax.ShapeDtypeStruct((B * ni, TI, 1), jnp.int32),
        ],
        interpret=_INTERP,
    )(m3, m3, n3, n3)
    node_max = out[0].reshape(B, T2)
    node_idx = out[1].reshape(B, T2)
    return node_max, node_idx


def _rank_body(ni2, TI2, c_ref, r_ref, o_ref):
    i = pl.program_id(1)
    vi = c_ref[0]
    vj = r_ref[0]
    gt = (vj > vi)
    jj = jax.lax.broadcasted_iota(jnp.int32, (TI2, T2), 1)
    ii = jax.lax.broadcasted_iota(jnp.int32, (TI2, T2), 0) + i * TI2
    tie = (vj == vi) & (jj < ii)
    rank = jnp.sum(gt.astype(jnp.int32) + tie.astype(jnp.int32),
                   axis=-1, keepdims=True)
    o_ref[...] = rank[None]


def _rank_tc(node_max):
    TI2 = 128
    ni2 = T2 // TI2
    nm_c = node_max.reshape(B * ni2, TI2, 1)
    nm_r = node_max.reshape(B, 1, T2)
    rank = pl.pallas_call(
        functools.partial(_rank_body, ni2, TI2),
        grid=(B, ni2),
        in_specs=[
            pl.BlockSpec((1, TI2, 1), lambda b, i: (b * ni2 + i, 0, 0)),
            pl.BlockSpec((1, 1, T2), lambda b, i: (b, 0, 0)),
        ],
        out_specs=pl.BlockSpec((1, TI2, 1), lambda b, i: (b * ni2 + i, 0, 0)),
        out_shape=jax.ShapeDtypeStruct((B * ni2, TI2, 1), jnp.int32),
        interpret=_INTERP,
    )(nm_c, nm_r)
    return rank.reshape(B, T2)


def _merge_body(rank_hbm, nidx_hbm, xq_hbm, out_hbm,
                rank_v, nidx_v, edge_v, edgeu_v, cnt_v,
                dstall_v, srcall_v, mpos_v, msrc_v, midx_v,
                uidx_v, widx_v, didx_v, urows_v, dbuf_v, mrows_v, sem):
    c = lax.axis_index("c")
    s = lax.axis_index("s")
    lanes = lax.iota(jnp.int32, 16)
    ones16 = jnp.ones((16,), jnp.float32)
    r0 = s * 128

    for bi in range(2):
        b = c * 2 + bi
        pltpu.sync_copy(rank_hbm.at[b], rank_v)
        pltpu.sync_copy(nidx_hbm.at[b], nidx_v)

        def inv_body(p, carry):
            rk = rank_v[pl.ds(p * 16, 16)]
            iv = p * 16 + lanes
            plsc.store_scatter(edge_v, [rk], iv, mask=rk < 208)
            plsc.store_scatter(edgeu_v, [rk - K], iv, mask=rk >= K)
            cnt_v[pl.ds(p * 16, 16)] = jnp.zeros((16,), jnp.float32)
            return carry

        lax.fori_loop(0, T2 // 16, inv_body, 0)

        for q in range(13):
            e = edge_v[pl.ds(q * 16, 16)]
            dpos = plsc.load_gather(nidx_v, [e])
            msk = (q * 16 + lanes) < K
            plsc.addupdate_scatter(cnt_v, [dpos], ones16, mask=msk)
            dstall_v[pl.ds(q * 16, 16)] = jnp.where(msk, dpos, T2)
            srcall_v[pl.ds(q * 16, 16)] = (b * T + 2 * e) * 4

        def scan_body(m, nm):
            dsp = plsc.load_gather(dstall_v, [jnp.full((16,), m, jnp.int32)])
            ssp = plsc.load_gather(srcall_v, [jnp.full((16,), m, jnp.int32)])
            dsc = jnp.sum(jnp.where(lanes == 0, dsp, 0))
            inwin = (dsc >= r0) & (dsc < r0 + 128)
            mk = (lanes == 0) & inwin
            plsc.store_scatter(mpos_v, [jnp.full((16,), nm, jnp.int32)], dsp,
                               mask=mk)
            plsc.store_scatter(msrc_v, [jnp.full((16,), nm, jnp.int32)], ssp,
                               mask=mk)
            return nm + inwin.astype(jnp.int32)

        nmatch = lax.fori_loop(0, K, scan_body, jnp.int32(0))
        plsc.store_scatter(msrc_v, [nmatch + lanes],
                           jnp.zeros((16,), jnp.int32))
        plsc.store_scatter(mpos_v, [nmatch + lanes],
                           jnp.full((16,), r0, jnp.int32))

        for it in range(8):
            t = it * 16 + s

            @pl.when(t < 116)
            def _():
                for q in range(4):
                    p = t * 16 + (q * 16 + lanes) // 4
                    pc = jnp.minimum(p, UNM - 1)
                    e = plsc.load_gather(edgeu_v, [pc])
                    quarter = lanes % 4
                    uidx_v[pl.ds(q * 16, 16)] = (b * T + 2 * e) * 4 + quarter
                    widx_v[pl.ds(q * 16, 16)] = (pc * B + b) * 4 + quarter
                pltpu.async_copy(xq_hbm.at[uidx_v], urows_v, sem).wait()
                pltpu.async_copy(urows_v, out_hbm.at[widx_v], sem).wait()

        for h in range(4):
            for q in range(8):
                j = r0 + q * 16 + lanes
                didx_v[pl.ds(q * 16, 16)] = (b * T + 2 * j + 1) * 4 + h
            pltpu.async_copy(xq_hbm.at[didx_v], dbuf_v, sem).wait()

            def shift_body(cq, carry):
                midx_v[pl.ds(cq * 16, 16)] = (
                    msrc_v[pl.ds(cq * 16, 16)] + h)
                return carry

            lax.fori_loop(0, 16, shift_body, 0)

            def grp_body(g, carry):
                pltpu.async_copy(
                    xq_hbm.at[midx_v.at[pl.ds(g * 16, 16)]], mrows_v,
                    sem).wait()

                def lane_body(l, carry2):
                    kk = g * 16 + l

                    @pl.when(kk < nmatch)
                    def _():
                        psp = plsc.load_gather(
                            mpos_v, [jnp.full((16,), kk, jnp.int32)])
                        row = jnp.sum(jnp.where(lanes == 0, psp, 0)) - r0
                        for cc in range(16):
                            dbuf_v[row, pl.ds(cc * 16, 16)] = (
                                dbuf_v[row, pl.ds(cc * 16, 16)]
                                + mrows_v[l, pl.ds(cc * 16, 16)])
                    return carry2

                lax.fori_loop(0, 16, lane_body, 0)
                return carry

            ngrp = (nmatch + 15) // 16
            lax.fori_loop(0, ngrp, grp_body, 0)

            def div_body(j, carry):
                csp = plsc.load_gather(
                    cnt_v, [jnp.full((16,), r0 + j, jnp.int32)])
                d = csp + 1.0
                for cc in range(16):
                    dbuf_v[j, pl.ds(cc * 16, 16)] = (
                        dbuf_v[j, pl.ds(cc * 16, 16)] / d)
                return carry

            lax.fori_loop(0, 128, div_body, 0)
            for q in range(8):
                j = r0 + q * 16 + lanes
                widx8 = ((UNM + j) * B + b) * 4 + h
                didx_v[pl.ds(q * 16, 16)] = widx8
            pltpu.async_copy(dbuf_v, out_hbm.at[didx_v], sem).wait()


def _merge_sc(rank, node_idx, x):
    xq = x.reshape(B * T * 4, C // 4)
    mesh = plsc.VectorSubcoreMesh(core_axis_name="c", subcore_axis_name="s")
    out = pl.kernel(
        _merge_body,
        out_type=jax.ShapeDtypeStruct((TOUT * B * 4, C // 4), jnp.float32),
        mesh=mesh,
        scratch_types=[
            pltpu.VMEM((T2,), jnp.int32),
            pltpu.VMEM((T2,), jnp.int32),
            pltpu.VMEM((256,), jnp.int32),
            pltpu.VMEM((1920,), jnp.int32),
            pltpu.VMEM((T2,), jnp.float32),
            pltpu.VMEM((208,), jnp.int32),
            pltpu.VMEM((208,), jnp.int32),
            pltpu.VMEM((256,), jnp.int32),
            pltpu.VMEM((256,), jnp.int32),
            pltpu.VMEM((256,), jnp.int32),
            pltpu.VMEM((64,), jnp.int32),
            pltpu.VMEM((64,), jnp.int32),
            pltpu.VMEM((128,), jnp.int32),
            pltpu.VMEM((64, 256), jnp.float32),
            pltpu.VMEM((128, 256), jnp.float32),
            pltpu.VMEM((16, 256), jnp.float32),
            pltpu.SemaphoreType.DMA,
        ],
        compiler_params=pltpu.CompilerParams(needs_layout_passes=False),
    )(rank, node_idx, xq)
    return out.reshape(TOUT, B, C).transpose(1, 0, 2)


def kernel(metric, x):
    node_max, node_idx = _node_max_idx(metric)
    rank = _rank_tc(node_max)
    return _merge_sc(rank, node_idx, x)
